# baseline (speedup 1.0000x reference)
import functools

import jax
import jax.numpy as jnp
from jax import lax
from jax.experimental import pallas as pl
from jax.experimental.pallas import tpu as pltpu
from jax.experimental.pallas import tpu_sc as plsc

N = 10000
E = 320000
D = 128
H = 8
C = D // H
K = 4

BN = 400
NB = N // BN
CH = 128
NCH = E // CH
CPT = 157
CH80 = 80
NCH80 = E // CH80
CPT80 = 250
RPT = 624
TAIL = N - 16 * RPT

_f32 = jnp.float32


@functools.cache
def _mesh():
    return plsc.VectorSubcoreMesh(core_axis_name="c", subcore_axis_name="s",
                                  num_cores=2, num_subcores=16)


def _sc_kernel(out_type, scratch_types):
    def deco(body):
        @functools.cache
        def build():
            return pl.kernel(body, out_type, mesh=_mesh(),
                             scratch_types=scratch_types)

        def call(*args):
            return build()(*args)

        return call

    return deco


def _chunk_range(w):
    start = w * CPT
    end = jnp.minimum(start + CPT, NCH)
    return start, end


def _chunk_range80(w):
    start = w * CPT80
    return start, start + CPT80


def _zero_acc(zsrc, acc, w):
    pltpu.sync_copy(zsrc, acc.at[pl.ds(w * RPT, RPT)])

    @pl.when(w == 0)
    def _():
        pltpu.sync_copy(zsrc.at[pl.ds(0, TAIL)], acc.at[pl.ds(16 * RPT, TAIL)])


def _writeout(acc, out, t, w):
    pltpu.sync_copy(acc.at[pl.ds(w * RPT, RPT)], out.at[t, pl.ds(w * RPT, RPT)])

    @pl.when(w == 0)
    def _():
        pltpu.sync_copy(acc.at[pl.ds(16 * RPT, TAIL)],
                        out.at[t, pl.ds(16 * RPT, TAIL)])


@_sc_kernel(
    jax.ShapeDtypeStruct((K, N, D), _f32),
    [
        pltpu.VMEM((3, CH), jnp.int32),
        pltpu.VMEM((CH, D), _f32),
        pltpu.VMEM_SHARED((N, D), _f32),
        pltpu.SemaphoreType.DMA((3,)),
        pltpu.SemaphoreType.DMA((2,)),
    ],
)
def _sc_degree(edges_r, zerosd, onesd, deg_out, didx, ones_v, acc, isem,
               ssem):
    c = lax.axis_index("c")
    w = lax.axis_index("s")
    pltpu.sync_copy(onesd, ones_v)
    start, end = _chunk_range(w)
    cnt = end - start
    for tt in (0, 1):
        t = c + 2 * tt
        _zero_acc(zerosd, acc, w)
        plsc.subcore_barrier()

        def idx_issue(j, t=t):
            p3 = j % 3
            pltpu.async_copy(edges_r.at[t, 1, start + j], didx.at[p3],
                             isem.at[p3])

        idx_issue(0)

        def body(j, _, t=t):
            p = j % 2
            p3 = j % 3

            @pl.when(j < cnt)
            def _():
                pltpu.make_async_copy(edges_r.at[t, 1, start + j],
                                      didx.at[p3], isem.at[p3]).wait()

            @pl.when(j >= 2)
            def _():
                pltpu.make_async_copy(ones_v, acc.at[didx.at[(j - 2) % 3]],
                                      ssem.at[p]).wait()

            @pl.when(j < cnt)
            def _():
                pltpu.async_copy(ones_v, acc.at[didx.at[p3]], ssem.at[p],
                                 add=True)

            @pl.when(j + 1 < cnt)
            def _():
                idx_issue(j + 1)

            return ()

        lax.fori_loop(0, cnt + 2, body, ())
        plsc.subcore_barrier()
        _writeout(acc, deg_out, t, w)
        plsc.subcore_barrier()


@_sc_kernel(
    jax.ShapeDtypeStruct((K, N, D), _f32),
    [
        pltpu.VMEM((3, CH), jnp.int32),
        pltpu.VMEM((3, CH), jnp.int32),
        pltpu.VMEM((2, CH, D), _f32),
        pltpu.VMEM_SHARED((N, D), _f32),
        pltpu.SemaphoreType.DMA((3,)),
        pltpu.SemaphoreType.DMA((2,)),
        pltpu.SemaphoreType.DMA((2,)),
    ],
)
def _sc_gcn_agg(mp, edges_r, zerosd, agg_out, sidx, didx, rows, acc,
                isem, gsem, ssem):
    c = lax.axis_index("c")
    w = lax.axis_index("s")
    start, end = _chunk_range(w)
    cnt = end - start
    for tt in (0, 1):
        t = c + 2 * tt
        _zero_acc(zerosd, acc, w)
        plsc.subcore_barrier()

        def idx_issue(j, t=t):
            p3 = j % 3
            ci = start + j
            pltpu.async_copy(edges_r.at[t, 0, ci], sidx.at[p3], isem.at[p3])
            pltpu.async_copy(edges_r.at[t, 1, ci], didx.at[p3], isem.at[p3])

        def idx_wait(j, t=t):
            p3 = j % 3
            ci = start + j
            pltpu.make_async_copy(edges_r.at[t, 0, ci], sidx.at[p3],
                                  isem.at[p3]).wait()
            pltpu.make_async_copy(edges_r.at[t, 1, ci], didx.at[p3],
                                  isem.at[p3]).wait()

        idx_issue(0)

        def body(j, _, t=t):
            p = j % 2
            p1 = (j + 1) % 2
            p3 = j % 3

            @pl.when(j < cnt)
            def _():
                idx_wait(j)

            @pl.when(j >= 2)
            def _():
                pltpu.make_async_copy(
                    rows.at[p], acc.at[didx.at[(j - 2) % 3]],
                    ssem.at[p]).wait()

            @pl.when(j < cnt)
            def _():
                pltpu.async_copy(mp.at[t].at[sidx.at[p3]], rows.at[p],
                                 gsem.at[p])

            @pl.when((j >= 1) & (j <= cnt))
            def _():
                pm3 = (j - 1) % 3
                pltpu.make_async_copy(
                    mp.at[t].at[sidx.at[pm3]], rows.at[p1],
                    gsem.at[p1]).wait()
                pltpu.async_copy(rows.at[p1], acc.at[didx.at[pm3]],
                                 ssem.at[p1], add=True)

            @pl.when(j + 1 < cnt)
            def _():
                idx_issue(j + 1)

            return ()

        lax.fori_loop(0, cnt + 2, body, ())
        plsc.subcore_barrier()
        _writeout(acc, agg_out, t, w)
        plsc.subcore_barrier()


@_sc_kernel(
    (
        jax.ShapeDtypeStruct((K, N, D), _f32),
        jax.ShapeDtypeStruct((K, E, D), _f32),
    ),
    [
        pltpu.VMEM((CH,), jnp.int32),
        pltpu.VMEM((CH,), jnp.int32),
        pltpu.VMEM((CH, D), _f32),
        pltpu.VMEM((CH, D), _f32),
        pltpu.VMEM((CH, D), _f32),
        pltpu.VMEM_SHARED((N, D), _f32),
    ],
)
def _sc_gat1(als, ald, edges_r, zerosd, den_out, ee_out,
             sidx, didx, asb, adb, eeb, acc):
    c = lax.axis_index("c")
    w = lax.axis_index("s")
    start, end = _chunk_range(w)
    for tt in (0, 1):
        t = c + 2 * tt
        _zero_acc(zerosd, acc, w)
        plsc.subcore_barrier()

        def body(ci, _, t=t):
            pltpu.sync_copy(edges_r.at[t, 0, ci], sidx)
            pltpu.sync_copy(edges_r.at[t, 1, ci], didx)
            pltpu.sync_copy(als.at[t].at[sidx], asb)
            pltpu.sync_copy(ald.at[t].at[didx], adb)

            def row(r, _):
                for h in range(H):
                    sl = pl.ds(16 * h, 16)
                    x = asb[r, sl] + adb[r, sl]
                    eeb[r, sl] = jnp.exp(jnp.maximum(x, 0.2 * x))
                return ()

            lax.fori_loop(0, CH, row, ())
            pltpu.sync_copy(eeb, acc.at[didx], add=True)
            pltpu.sync_copy(eeb, ee_out.at[t, pl.ds(ci * CH, CH)])
            return ()

        lax.fori_loop(start, end, body, ())
        plsc.subcore_barrier()
        _writeout(acc, den_out, t, w)
        plsc.subcore_barrier()


@_sc_kernel(
    jax.ShapeDtypeStruct((K, N, D), _f32),
    [
        pltpu.VMEM((3, CH80), jnp.int32),
        pltpu.VMEM((3, CH80), jnp.int32),
        pltpu.VMEM((2, CH80, D), _f32),
        pltpu.VMEM((2, CH80, D), _f32),
        pltpu.VMEM_SHARED((N, D), _f32),
        pltpu.SemaphoreType.DMA((3,)),
        pltpu.SemaphoreType.DMA((2,)),
        pltpu.SemaphoreType.DMA((2,)),
    ],
)
def _sc_gat2(hm, ee, edges_r, zerosd, gacc_out,
             sidx, didx, hbuf, eeb, acc, isem, gsem, ssem):
    c = lax.axis_index("c")
    w = lax.axis_index("s")
    start, end = _chunk_range80(w)
    cnt = end - start
    for tt in (0, 1):
        t = c + 2 * tt
        _zero_acc(zerosd, acc, w)
        plsc.subcore_barrier()

        def idx_issue(j, t=t):
            p3 = j % 3
            ci = start + j
            pltpu.async_copy(edges_r.at[t, 0, ci], sidx.at[p3], isem.at[p3])
            pltpu.async_copy(edges_r.at[t, 1, ci], didx.at[p3], isem.at[p3])

        idx_issue(0)

        def body(j, _, t=t):
            p = j % 2
            p1 = (j + 1) % 2
            p3 = j % 3

            @pl.when(j < cnt)
            def _():
                ci = start + j
                pltpu.make_async_copy(edges_r.at[t, 0, ci], sidx.at[p3],
                                      isem.at[p3]).wait()
                pltpu.make_async_copy(edges_r.at[t, 1, ci], didx.at[p3],
                                      isem.at[p3]).wait()

            @pl.when(j >= 2)
            def _():
                pm3 = (j - 2) % 3
                pltpu.make_async_copy(eeb.at[p], acc.at[didx.at[pm3]],
                                      ssem.at[p]).wait()

            @pl.when(j < cnt)
            def _():
                ci = start + j
                pltpu.async_copy(hm.at[t].at[sidx.at[p3]], hbuf.at[p],
                                 gsem.at[p])
                pltpu.async_copy(ee.at[t, pl.ds(ci * CH80, CH80)], eeb.at[p],
                                 gsem.at[p])

            @pl.when((j >= 1) & (j <= cnt))
            def _():
                pm3 = (j - 1) % 3
                ci = start + j - 1
                pltpu.make_async_copy(hm.at[t].at[sidx.at[pm3]], hbuf.at[p1],
                                      gsem.at[p1]).wait()
                pltpu.make_async_copy(ee.at[t, pl.ds(ci * CH80, CH80)],
                                      eeb.at[p1], gsem.at[p1]).wait()

                def row(r, _):
                    for h in range(H):
                        sl = pl.ds(16 * h, 16)
                        eeb[p1, r, sl] = eeb[p1, r, sl] * hbuf[p1, r, sl]
                    return ()

                lax.fori_loop(0, CH80, row, ())
                pltpu.async_copy(eeb.at[p1], acc.at[didx.at[pm3]],
                                 ssem.at[p1], add=True)

            @pl.when(j + 1 < cnt)
            def _():
                idx_issue(j + 1)

            return ()

        lax.fori_loop(0, cnt + 2, body, ())
        plsc.subcore_barrier()
        _writeout(acc, gacc_out, t, w)
        plsc.subcore_barrier()


def _dinv_of(deg_ref):
    return lax.rsqrt(deg_ref[0, :, 0:1] + 1.0)


def _tc0_body(x_ref, w_ref, deg_ref, m_ref, mp_ref):
    m = jnp.dot(x_ref[...], w_ref[0].T, preferred_element_type=_f32)
    dinv = _dinv_of(deg_ref)
    m_ref[0] = m
    mp_ref[0] = m * dinv


def _tc_layer_body(agg_ref, mprev_ref, deg_ref, b_ref, w_ref, m_ref, mp_ref):
    dinv = _dinv_of(deg_ref)
    h = jnp.maximum(dinv * agg_ref[0] + dinv * dinv * mprev_ref[0] + b_ref[0],
                    0.0)
    m = jnp.dot(h, w_ref[0].T, preferred_element_type=_f32)
    m_ref[0] = m
    mp_ref[0] = m * dinv


def _tc_gatprep_body(agg_ref, mprev_ref, deg_ref, b_ref, w_ref, asx_ref,
                     adx_ref, hm_ref, als_ref, ald_ref, ees_ref):
    dinv = _dinv_of(deg_ref)
    h = jnp.maximum(dinv * agg_ref[0] + dinv * dinv * mprev_ref[0] + b_ref[0],
                    0.0)
    hmv = jnp.dot(h, w_ref[0].T, preferred_element_type=_f32)
    als = jnp.dot(hmv, asx_ref[0], preferred_element_type=_f32)
    ald = jnp.dot(hmv, adx_ref[0], preferred_element_type=_f32)
    x = als + ald
    ee = jnp.exp(jnp.maximum(x, 0.2 * x))
    hm_ref[0] = hmv
    als_ref[0] = als
    ald_ref[0] = ald
    ees_ref[0] = ee


def _tc_deninv_body(den_ref, ees_ref, out_ref):
    out_ref[0] = 1.0 / (den_ref[0] + ees_ref[0] + 1e-16)


def _tc_final_body(gacc_ref, hm_ref, ees_ref, din_ref, gb_ref, lng_ref,
                   lnb_ref, sty_ref, strs_ref, w1a_ref, w1b_ref,
                   b1_ref, w2_ref, b2_ref, out_ref):
    a = jnp.tanh(jnp.dot(sty_ref[...], w1a_ref[...].T,
                         preferred_element_type=_f32)
                 + jnp.dot(strs_ref[...], w1b_ref[...].T,
                           preferred_element_type=_f32)
                 + b1_ref[...])
    logits = jnp.dot(a, w2_ref[...], preferred_element_type=_f32) + b2_ref[...]
    mx = jnp.max(logits, axis=-1, keepdims=True)
    ew = jnp.exp(logits - mx)
    wgt = ew / jnp.sum(ew, axis=-1, keepdims=True)
    acc = jnp.zeros((BN, D), _f32)
    for k in range(K):
        g = (din_ref[k] * (gacc_ref[k] + ees_ref[k] * hm_ref[k])
             + gb_ref[k])
        mu = jnp.mean(g, axis=-1, keepdims=True)
        var = jnp.mean((g - mu) ** 2, axis=-1, keepdims=True)
        hk = (g - mu) * lax.rsqrt(var + 1e-5) * lng_ref[k] + lnb_ref[k]
        acc = acc + wgt[:, k:k + 1] * hk
    out_ref[...] = acc


def _nd(shape):
    return jax.ShapeDtypeStruct(shape, _f32)


_B_KND = pl.BlockSpec((1, BN, D), lambda k, i: (k, i, 0))
_B_KDD = pl.BlockSpec((1, D, D), lambda k, i: (k, 0, 0))
_B_K1D = pl.BlockSpec((1, 1, D), lambda k, i: (k, 0, 0))


def _tc0(x, w0, deg):
    return pl.pallas_call(
        _tc0_body,
        grid=(K, NB),
        in_specs=[pl.BlockSpec((BN, D), lambda k, i: (i, 0)), _B_KDD, _B_KND],
        out_specs=[_B_KND, _B_KND],
        out_shape=[_nd((K, N, D)), _nd((K, N, D))],
    )(x, w0, deg)


def _tc_layer(agg, mprev, deg, b, w):
    return pl.pallas_call(
        _tc_layer_body,
        grid=(K, NB),
        in_specs=[_B_KND, _B_KND, _B_KND, _B_K1D, _B_KDD],
        out_specs=[_B_KND, _B_KND],
        out_shape=[_nd((K, N, D)), _nd((K, N, D))],
    )(agg, mprev, deg, b, w)


def _tc_gatprep(agg, mprev, deg, b, w, asx, adx):
    return pl.pallas_call(
        _tc_gatprep_body,
        grid=(K, NB),
        in_specs=[_B_KND, _B_KND, _B_KND, _B_K1D, _B_KDD, _B_KDD, _B_KDD],
        out_specs=[_B_KND, _B_KND, _B_KND, _B_KND],
        out_shape=[_nd((K, N, D)), _nd((K, N, D)), _nd((K, N, D)),
                   _nd((K, N, D))],
    )(agg, mprev, deg, b, w, asx, adx)


def _tc_deninv(den, ees):
    return pl.pallas_call(
        _tc_deninv_body,
        grid=(K, NB),
        in_specs=[_B_KND, _B_KND],
        out_specs=_B_KND,
        out_shape=_nd((K, N, D)),
    )(den, ees)


def _tc_final(gacc, hm, ees, din, gat_b, ln_g, ln_b, style, stress,
              w1a, w1b, b1, w2p, b2p):
    full = lambda *s: pl.BlockSpec(s, lambda i: (0,) * len(s))
    return pl.pallas_call(
        _tc_final_body,
        grid=(NB,),
        in_specs=[
            pl.BlockSpec((K, BN, D), lambda i: (0, i, 0)),
            pl.BlockSpec((K, BN, D), lambda i: (0, i, 0)),
            pl.BlockSpec((K, BN, D), lambda i: (0, i, 0)),
            pl.BlockSpec((K, BN, D), lambda i: (0, i, 0)),
            full(K, D), full(K, D), full(K, D),
            pl.BlockSpec((BN, D), lambda i: (i, 0)),
            pl.BlockSpec((BN, D), lambda i: (i, 0)),
            full(D, D), full(D, D), full(D), full(D, 8), full(8),
        ],
        out_specs=pl.BlockSpec((BN, D), lambda i: (i, 0)),
        out_shape=_nd((N, D)),
    )(gacc, hm, ees, din, gat_b, ln_g, ln_b, style, stress,
      w1a, w1b, b1, w2p, b2p)


def kernel(x, edge_index_list, style_features, stress_features, gcn_W, gcn_b,
           gat_W, gat_att_src, gat_att_dst, gat_b, ln_g, ln_b, fus_W1, fus_b1,
           fus_W2, fus_b2):
    edges_r = edge_index_list.reshape(K, 2, NCH, CH)
    edges_r80 = edge_index_list.reshape(K, 2, NCH80, CH80)
    zerosd = jnp.zeros((RPT, D), _f32)
    onesd = jnp.ones((CH, D), _f32)

    rep = jnp.kron(jnp.eye(H, dtype=_f32), jnp.ones((1, C), _f32))
    asbd = jnp.einsum('khc,hg->khcg', gat_att_src,
                      jnp.eye(H, dtype=_f32)).reshape(K, D, H)
    adbd = jnp.einsum('khc,hg->khcg', gat_att_dst,
                      jnp.eye(H, dtype=_f32)).reshape(K, D, H)
    asx = jnp.einsum('kdh,hg->kdg', asbd, rep)
    adx = jnp.einsum('kdh,hg->kdg', adbd, rep)
    w1a = fus_W1[:, :D]
    w1b = fus_W1[:, D:]
    w2p = jnp.pad(fus_W2.T, ((0, 0), (0, 8 - K)))
    b2p = jnp.concatenate([fus_b2, jnp.full((8 - K,), -1e30, _f32)])

    deg = _sc_degree(edges_r, zerosd, onesd)

    m, mp = _tc0(x, gcn_W[:, 0], deg)
    for l in (1, 2):
        agg = _sc_gcn_agg(mp, edges_r, zerosd)
        m, mp = _tc_layer(agg, m, deg, gcn_b[:, l - 1][:, None, :],
                          gcn_W[:, l])
    agg = _sc_gcn_agg(mp, edges_r, zerosd)
    hm, als, ald, ees = _tc_gatprep(agg, m, deg, gcn_b[:, 2][:, None, :],
                                    gat_W, asx, adx)
    den, ee = _sc_gat1(als, ald, edges_r, zerosd)
    din = _tc_deninv(den, ees)
    gacc = _sc_gat2(hm, ee, edges_r80, zerosd)
    return _tc_final(gacc, hm, ees, din, gat_b, ln_g, ln_b,
                     style_features, stress_features, w1a, w1b, fus_b1,
                     w2p, b2p)


# speedup vs baseline: 1.4629x; 1.4629x over previous
import functools

import jax
import jax.numpy as jnp
from jax import lax
from jax.experimental import pallas as pl
from jax.experimental.pallas import tpu as pltpu
from jax.experimental.pallas import tpu_sc as plsc

N = 10000
E = 320000
D = 128
H = 8
C = D // H
K = 4

BN = 400
NB = N // BN
CH = 128
NCH = E // CH
CPT = 157
CH80 = 80
NCH80 = E // CH80
CPT80 = 250
RPT = 624
TAIL = N - 16 * RPT

_f32 = jnp.float32


@functools.cache
def _mesh():
    return plsc.VectorSubcoreMesh(core_axis_name="c", subcore_axis_name="s",
                                  num_cores=2, num_subcores=16)


def _sc_kernel(out_type, scratch_types):
    def deco(body):
        @functools.cache
        def build():
            return pl.kernel(body, out_type, mesh=_mesh(),
                             scratch_types=scratch_types)

        def call(*args):
            return build()(*args)

        return call

    return deco


def _chunk_range(w):
    start = w * CPT
    end = jnp.minimum(start + CPT, NCH)
    return start, end


def _chunk_range80(w):
    start = w * CPT80
    return start, start + CPT80


def _zero_acc(zsrc, acc, w):
    pltpu.sync_copy(zsrc, acc.at[pl.ds(w * RPT, RPT)])

    @pl.when(w == 0)
    def _():
        pltpu.sync_copy(zsrc.at[pl.ds(0, TAIL)], acc.at[pl.ds(16 * RPT, TAIL)])


def _writeout(acc, out, t, w):
    pltpu.sync_copy(acc.at[pl.ds(w * RPT, RPT)], out.at[t, pl.ds(w * RPT, RPT)])

    @pl.when(w == 0)
    def _():
        pltpu.sync_copy(acc.at[pl.ds(16 * RPT, TAIL)],
                        out.at[t, pl.ds(16 * RPT, TAIL)])


@_sc_kernel(
    jax.ShapeDtypeStruct((K, N, D), _f32),
    [
        pltpu.VMEM((3, CH), jnp.int32),
        pltpu.VMEM((CH, D), _f32),
        pltpu.VMEM_SHARED((N, D), _f32),
        pltpu.SemaphoreType.DMA((3,)),
        pltpu.SemaphoreType.DMA((2,)),
    ],
)
def _sc_degree(edges_r, zerosd, onesd, deg_out, didx, ones_v, acc, isem,
               ssem):
    c = lax.axis_index("c")
    w = lax.axis_index("s")
    pltpu.sync_copy(onesd, ones_v)
    start, end = _chunk_range(w)
    cnt = end - start
    for tt in (0, 1):
        t = c + 2 * tt
        _zero_acc(zerosd, acc, w)
        plsc.subcore_barrier()

        def idx_issue(j, t=t):
            p3 = j % 3
            pltpu.async_copy(edges_r.at[t, 1, start + j], didx.at[p3],
                             isem.at[p3])

        idx_issue(0)

        def body(j, _, t=t):
            p = j % 2
            p3 = j % 3

            @pl.when(j < cnt)
            def _():
                pltpu.make_async_copy(edges_r.at[t, 1, start + j],
                                      didx.at[p3], isem.at[p3]).wait()

            @pl.when(j >= 2)
            def _():
                pltpu.make_async_copy(ones_v, acc.at[didx.at[(j - 2) % 3]],
                                      ssem.at[p]).wait()

            @pl.when(j < cnt)
            def _():
                pltpu.async_copy(ones_v, acc.at[didx.at[p3]], ssem.at[p],
                                 add=True)

            @pl.when(j + 1 < cnt)
            def _():
                idx_issue(j + 1)

            return ()

        lax.fori_loop(0, cnt + 2, body, ())
        plsc.subcore_barrier()
        _writeout(acc, deg_out, t, w)
        plsc.subcore_barrier()


@_sc_kernel(
    jax.ShapeDtypeStruct((K, N, D), _f32),
    [
        pltpu.VMEM((3, CH), jnp.int32),
        pltpu.VMEM((3, CH), jnp.int32),
        pltpu.VMEM((2, CH, D), _f32),
        pltpu.VMEM_SHARED((N, D), _f32),
        pltpu.SemaphoreType.DMA((3,)),
        pltpu.SemaphoreType.DMA((2,)),
        pltpu.SemaphoreType.DMA((2,)),
    ],
)
def _sc_gcn_agg(mp, edges_r, zerosd, agg_out, sidx, didx, rows, acc,
                isem, gsem, ssem):
    c = lax.axis_index("c")
    w = lax.axis_index("s")
    start, end = _chunk_range(w)
    cnt = end - start
    for tt in (0, 1):
        t = c + 2 * tt
        _zero_acc(zerosd, acc, w)
        plsc.subcore_barrier()

        def idx_issue(j, t=t):
            p3 = j % 3
            ci = start + j
            pltpu.async_copy(edges_r.at[t, 0, ci], sidx.at[p3], isem.at[p3])
            pltpu.async_copy(edges_r.at[t, 1, ci], didx.at[p3], isem.at[p3])

        def idx_wait(j, t=t):
            p3 = j % 3
            ci = start + j
            pltpu.make_async_copy(edges_r.at[t, 0, ci], sidx.at[p3],
                                  isem.at[p3]).wait()
            pltpu.make_async_copy(edges_r.at[t, 1, ci], didx.at[p3],
                                  isem.at[p3]).wait()

        idx_issue(0)

        def body(j, _, t=t):
            p = j % 2
            p1 = (j + 1) % 2
            p3 = j % 3

            @pl.when(j < cnt)
            def _():
                idx_wait(j)

            @pl.when(j >= 2)
            def _():
                pltpu.make_async_copy(
                    rows.at[p], acc.at[didx.at[(j - 2) % 3]],
                    ssem.at[p]).wait()

            @pl.when(j < cnt)
            def _():
                pltpu.async_copy(mp.at[t].at[sidx.at[p3]], rows.at[p],
                                 gsem.at[p])

            @pl.when((j >= 1) & (j <= cnt))
            def _():
                pm3 = (j - 1) % 3
                pltpu.make_async_copy(
                    mp.at[t].at[sidx.at[pm3]], rows.at[p1],
                    gsem.at[p1]).wait()
                pltpu.async_copy(rows.at[p1], acc.at[didx.at[pm3]],
                                 ssem.at[p1], add=True)

            @pl.when(j + 1 < cnt)
            def _():
                idx_issue(j + 1)

            return ()

        lax.fori_loop(0, cnt + 2, body, ())
        plsc.subcore_barrier()
        _writeout(acc, agg_out, t, w)
        plsc.subcore_barrier()


@_sc_kernel(
    (
        jax.ShapeDtypeStruct((K, N, D), _f32),
        jax.ShapeDtypeStruct((K, E, D), _f32),
    ),
    [
        pltpu.VMEM((2, CH80), jnp.int32),
        pltpu.VMEM((2, CH80), jnp.int32),
        pltpu.VMEM((CH80, D), _f32),
        pltpu.VMEM((CH80, D), _f32),
        pltpu.VMEM((CH80, D), _f32),
        pltpu.VMEM((CH80, D), _f32),
        pltpu.VMEM_SHARED((N, D), _f32),
        pltpu.SemaphoreType.DMA,
        pltpu.SemaphoreType.DMA,
        pltpu.SemaphoreType.DMA,
        pltpu.SemaphoreType.DMA,
        pltpu.SemaphoreType.DMA,
        pltpu.SemaphoreType.DMA,
    ],
)
def _sc_gat1(als, ald, edges_t, zerosd, den_out, ee_out,
             eba, ebb, asba, adba, asbb, adbb, acc,
             gsa, gsb, ssa, ssb, wsa, wsb):
    c = lax.axis_index("c")
    w = lax.axis_index("s")
    start, end = _chunk_range80(w)
    npairs = (end - start) // 2

    def compute_ee(asb, adb):
        def row(r, _):
            for h in range(H):
                sl = pl.ds(16 * h, 16)
                x = asb[r, sl] + adb[r, sl]
                adb[r, sl] = jnp.exp(jnp.maximum(x, 0.2 * x))
            return ()

        lax.fori_loop(0, CH80, row, ())

    for tt in (0, 1):
        t = c + 2 * tt
        _zero_acc(zerosd, acc, w)
        plsc.subcore_barrier()

        def body(i, _, t=t):
            cia = start + 2 * i
            cib = cia + 1

            @pl.when(i >= 1)
            def _():
                pltpu.make_async_copy(adba, acc.at[eba.at[1]], ssa).wait()
                pltpu.make_async_copy(
                    adba, ee_out.at[t, pl.ds((cia - 2) * CH80, CH80)],
                    wsa).wait()

            @pl.when(i < npairs)
            def _():
                pltpu.sync_copy(edges_t.at[t, cia], eba)
                pltpu.async_copy(als.at[t].at[eba.at[0]], asba, gsa)
                pltpu.async_copy(ald.at[t].at[eba.at[1]], adba, gsa)

            @pl.when(i >= 1)
            def _():
                pltpu.make_async_copy(adbb, acc.at[ebb.at[1]], ssb).wait()
                pltpu.make_async_copy(
                    adbb, ee_out.at[t, pl.ds((cib - 2) * CH80, CH80)],
                    wsb).wait()

            @pl.when(i < npairs)
            def _():
                pltpu.sync_copy(edges_t.at[t, cib], ebb)
                pltpu.async_copy(als.at[t].at[ebb.at[0]], asbb, gsb)
                pltpu.async_copy(ald.at[t].at[ebb.at[1]], adbb, gsb)

                pltpu.make_async_copy(als.at[t].at[eba.at[0]], asba,
                                      gsa).wait()
                pltpu.make_async_copy(ald.at[t].at[eba.at[1]], adba,
                                      gsa).wait()
                compute_ee(asba, adba)
                pltpu.async_copy(adba, acc.at[eba.at[1]], ssa, add=True)
                pltpu.async_copy(adba,
                                 ee_out.at[t, pl.ds(cia * CH80, CH80)], wsa)

                pltpu.make_async_copy(als.at[t].at[ebb.at[0]], asbb,
                                      gsb).wait()
                pltpu.make_async_copy(ald.at[t].at[ebb.at[1]], adbb,
                                      gsb).wait()
                compute_ee(asbb, adbb)
                pltpu.async_copy(adbb, acc.at[ebb.at[1]], ssb, add=True)
                pltpu.async_copy(adbb,
                                 ee_out.at[t, pl.ds(cib * CH80, CH80)], wsb)

            return ()

        lax.fori_loop(0, npairs + 1, body, ())
        plsc.subcore_barrier()
        _writeout(acc, den_out, t, w)
        plsc.subcore_barrier()


@_sc_kernel(
    jax.ShapeDtypeStruct((K, N, D), _f32),
    [
        pltpu.VMEM((2, CH80), jnp.int32),
        pltpu.VMEM((2, CH80), jnp.int32),
        pltpu.VMEM((CH80, D), _f32),
        pltpu.VMEM((CH80, D), _f32),
        pltpu.VMEM((CH80, D), _f32),
        pltpu.VMEM((CH80, D), _f32),
        pltpu.VMEM_SHARED((N, D), _f32),
        pltpu.SemaphoreType.DMA,
        pltpu.SemaphoreType.DMA,
        pltpu.SemaphoreType.DMA,
        pltpu.SemaphoreType.DMA,
    ],
)
def _sc_gat2(hm, ee, edges_t, zerosd, gacc_out,
             eba, ebb, hba, eeba, hbb, eebb, acc, gsa, gsb, ssa, ssb):
    c = lax.axis_index("c")
    w = lax.axis_index("s")
    start, end = _chunk_range80(w)
    npairs = (end - start) // 2

    def compute_mul(hb, eeb):
        def row(r, _):
            for h in range(H):
                sl = pl.ds(16 * h, 16)
                eeb[r, sl] = eeb[r, sl] * hb[r, sl]
            return ()

        lax.fori_loop(0, CH80, row, ())

    for tt in (0, 1):
        t = c + 2 * tt
        _zero_acc(zerosd, acc, w)
        plsc.subcore_barrier()

        def body(i, _, t=t):
            cia = start + 2 * i
            cib = cia + 1

            @pl.when(i >= 1)
            def _():
                pltpu.make_async_copy(eeba, acc.at[eba.at[1]], ssa).wait()

            @pl.when(i < npairs)
            def _():
                pltpu.sync_copy(edges_t.at[t, cia], eba)
                pltpu.async_copy(hm.at[t].at[eba.at[0]], hba, gsa)
                pltpu.async_copy(ee.at[t, pl.ds(cia * CH80, CH80)], eeba,
                                 gsa)

            @pl.when(i >= 1)
            def _():
                pltpu.make_async_copy(eebb, acc.at[ebb.at[1]], ssb).wait()

            @pl.when(i < npairs)
            def _():
                pltpu.sync_copy(edges_t.at[t, cib], ebb)
                pltpu.async_copy(hm.at[t].at[ebb.at[0]], hbb, gsb)
                pltpu.async_copy(ee.at[t, pl.ds(cib * CH80, CH80)], eebb,
                                 gsb)

                pltpu.make_async_copy(hm.at[t].at[eba.at[0]], hba,
                                      gsa).wait()
                pltpu.make_async_copy(ee.at[t, pl.ds(cia * CH80, CH80)],
                                      eeba, gsa).wait()
                compute_mul(hba, eeba)
                pltpu.async_copy(eeba, acc.at[eba.at[1]], ssa, add=True)

                pltpu.make_async_copy(hm.at[t].at[ebb.at[0]], hbb,
                                      gsb).wait()
                pltpu.make_async_copy(ee.at[t, pl.ds(cib * CH80, CH80)],
                                      eebb, gsb).wait()
                compute_mul(hbb, eebb)
                pltpu.async_copy(eebb, acc.at[ebb.at[1]], ssb, add=True)

            return ()

        lax.fori_loop(0, npairs + 1, body, ())
        plsc.subcore_barrier()
        _writeout(acc, gacc_out, t, w)
        plsc.subcore_barrier()


def _dinv_of(deg_ref):
    return lax.rsqrt(deg_ref[0, :, 0:1] + 1.0)


def _tc0_body(x_ref, w_ref, deg_ref, m_ref, mp_ref):
    m = jnp.dot(x_ref[...], w_ref[0].T, preferred_element_type=_f32)
    dinv = _dinv_of(deg_ref)
    m_ref[0] = m
    mp_ref[0] = m * dinv


def _tc_layer_body(agg_ref, mprev_ref, deg_ref, b_ref, w_ref, m_ref, mp_ref):
    dinv = _dinv_of(deg_ref)
    h = jnp.maximum(dinv * agg_ref[0] + dinv * dinv * mprev_ref[0] + b_ref[0],
                    0.0)
    m = jnp.dot(h, w_ref[0].T, preferred_element_type=_f32)
    m_ref[0] = m
    mp_ref[0] = m * dinv


def _tc_gatprep_body(agg_ref, mprev_ref, deg_ref, b_ref, w_ref, asx_ref,
                     adx_ref, hm_ref, als_ref, ald_ref, ees_ref):
    dinv = _dinv_of(deg_ref)
    h = jnp.maximum(dinv * agg_ref[0] + dinv * dinv * mprev_ref[0] + b_ref[0],
                    0.0)
    hmv = jnp.dot(h, w_ref[0].T, preferred_element_type=_f32)
    als = jnp.dot(hmv, asx_ref[0], preferred_element_type=_f32)
    ald = jnp.dot(hmv, adx_ref[0], preferred_element_type=_f32)
    x = als + ald
    ee = jnp.exp(jnp.maximum(x, 0.2 * x))
    hm_ref[0] = hmv
    als_ref[0] = als
    ald_ref[0] = ald
    ees_ref[0] = ee


def _tc_deninv_body(den_ref, ees_ref, out_ref):
    out_ref[0] = 1.0 / (den_ref[0] + ees_ref[0] + 1e-16)


def _tc_final_body(gacc_ref, hm_ref, ees_ref, din_ref, gb_ref, lng_ref,
                   lnb_ref, sty_ref, strs_ref, w1a_ref, w1b_ref,
                   b1_ref, w2_ref, b2_ref, out_ref):
    a = jnp.tanh(jnp.dot(sty_ref[...], w1a_ref[...].T,
                         preferred_element_type=_f32)
                 + jnp.dot(strs_ref[...], w1b_ref[...].T,
                           preferred_element_type=_f32)
                 + b1_ref[...])
    logits = jnp.dot(a, w2_ref[...], preferred_element_type=_f32) + b2_ref[...]
    mx = jnp.max(logits, axis=-1, keepdims=True)
    ew = jnp.exp(logits - mx)
    wgt = ew / jnp.sum(ew, axis=-1, keepdims=True)
    acc = jnp.zeros((BN, D), _f32)
    for k in range(K):
        g = (din_ref[k] * (gacc_ref[k] + ees_ref[k] * hm_ref[k])
             + gb_ref[k])
        mu = jnp.mean(g, axis=-1, keepdims=True)
        var = jnp.mean((g - mu) ** 2, axis=-1, keepdims=True)
        hk = (g - mu) * lax.rsqrt(var + 1e-5) * lng_ref[k] + lnb_ref[k]
        acc = acc + wgt[:, k:k + 1] * hk
    out_ref[...] = acc


def _nd(shape):
    return jax.ShapeDtypeStruct(shape, _f32)


_B_KND = pl.BlockSpec((1, BN, D), lambda k, i: (k, i, 0))
_B_KDD = pl.BlockSpec((1, D, D), lambda k, i: (k, 0, 0))
_B_K1D = pl.BlockSpec((1, 1, D), lambda k, i: (k, 0, 0))


def _tc0(x, w0, deg):
    return pl.pallas_call(
        _tc0_body,
        grid=(K, NB),
        in_specs=[pl.BlockSpec((BN, D), lambda k, i: (i, 0)), _B_KDD, _B_KND],
        out_specs=[_B_KND, _B_KND],
        out_shape=[_nd((K, N, D)), _nd((K, N, D))],
    )(x, w0, deg)


def _tc_layer(agg, mprev, deg, b, w):
    return pl.pallas_call(
        _tc_layer_body,
        grid=(K, NB),
        in_specs=[_B_KND, _B_KND, _B_KND, _B_K1D, _B_KDD],
        out_specs=[_B_KND, _B_KND],
        out_shape=[_nd((K, N, D)), _nd((K, N, D))],
    )(agg, mprev, deg, b, w)


def _tc_gatprep(agg, mprev, deg, b, w, asx, adx):
    return pl.pallas_call(
        _tc_gatprep_body,
        grid=(K, NB),
        in_specs=[_B_KND, _B_KND, _B_KND, _B_K1D, _B_KDD, _B_KDD, _B_KDD],
        out_specs=[_B_KND, _B_KND, _B_KND, _B_KND],
        out_shape=[_nd((K, N, D)), _nd((K, N, D)), _nd((K, N, D)),
                   _nd((K, N, D))],
    )(agg, mprev, deg, b, w, asx, adx)


def _tc_deninv(den, ees):
    return pl.pallas_call(
        _tc_deninv_body,
        grid=(K, NB),
        in_specs=[_B_KND, _B_KND],
        out_specs=_B_KND,
        out_shape=_nd((K, N, D)),
    )(den, ees)


def _tc_final(gacc, hm, ees, din, gat_b, ln_g, ln_b, style, stress,
              w1a, w1b, b1, w2p, b2p):
    full = lambda *s: pl.BlockSpec(s, lambda i: (0,) * len(s))
    return pl.pallas_call(
        _tc_final_body,
        grid=(NB,),
        in_specs=[
            pl.BlockSpec((K, BN, D), lambda i: (0, i, 0)),
            pl.BlockSpec((K, BN, D), lambda i: (0, i, 0)),
            pl.BlockSpec((K, BN, D), lambda i: (0, i, 0)),
            pl.BlockSpec((K, BN, D), lambda i: (0, i, 0)),
            full(K, D), full(K, D), full(K, D),
            pl.BlockSpec((BN, D), lambda i: (i, 0)),
            pl.BlockSpec((BN, D), lambda i: (i, 0)),
            full(D, D), full(D, D), full(D), full(D, 8), full(8),
        ],
        out_specs=pl.BlockSpec((BN, D), lambda i: (i, 0)),
        out_shape=_nd((N, D)),
    )(gacc, hm, ees, din, gat_b, ln_g, ln_b, style, stress,
      w1a, w1b, b1, w2p, b2p)


def kernel(x, edge_index_list, style_features, stress_features, gcn_W, gcn_b,
           gat_W, gat_att_src, gat_att_dst, gat_b, ln_g, ln_b, fus_W1, fus_b1,
           fus_W2, fus_b2):
    edges_r = edge_index_list.reshape(K, 2, NCH, CH)
    edges_t80 = edge_index_list.reshape(K, 2, NCH80, CH80).transpose(0, 2, 1, 3)
    zerosd = jnp.zeros((RPT, D), _f32)
    onesd = jnp.ones((CH, D), _f32)

    rep = jnp.kron(jnp.eye(H, dtype=_f32), jnp.ones((1, C), _f32))
    asbd = jnp.einsum('khc,hg->khcg', gat_att_src,
                      jnp.eye(H, dtype=_f32)).reshape(K, D, H)
    adbd = jnp.einsum('khc,hg->khcg', gat_att_dst,
                      jnp.eye(H, dtype=_f32)).reshape(K, D, H)
    asx = jnp.einsum('kdh,hg->kdg', asbd, rep)
    adx = jnp.einsum('kdh,hg->kdg', adbd, rep)
    w1a = fus_W1[:, :D]
    w1b = fus_W1[:, D:]
    w2p = jnp.pad(fus_W2.T, ((0, 0), (0, 8 - K)))
    b2p = jnp.concatenate([fus_b2, jnp.full((8 - K,), -1e30, _f32)])

    deg = _sc_degree(edges_r, zerosd, onesd)

    m, mp = _tc0(x, gcn_W[:, 0], deg)
    for l in (1, 2):
        agg = _sc_gcn_agg(mp, edges_r, zerosd)
        m, mp = _tc_layer(agg, m, deg, gcn_b[:, l - 1][:, None, :],
                          gcn_W[:, l])
    agg = _sc_gcn_agg(mp, edges_r, zerosd)
    hm, als, ald, ees = _tc_gatprep(agg, m, deg, gcn_b[:, 2][:, None, :],
                                    gat_W, asx, adx)
    den, ee = _sc_gat1(als, ald, edges_t80, zerosd)
    din = _tc_deninv(den, ees)
    gacc = _sc_gat2(hm, ee, edges_t80, zerosd)
    return _tc_final(gacc, hm, ees, din, gat_b, ln_g, ln_b,
                     style_features, stress_features, w1a, w1b, fus_b1,
                     w2p, b2p)


# speedup vs baseline: 1.4654x; 1.0017x over previous
import functools

import jax
import jax.numpy as jnp
from jax import lax
from jax.experimental import pallas as pl
from jax.experimental.pallas import tpu as pltpu
from jax.experimental.pallas import tpu_sc as plsc

N = 10000
E = 320000
D = 128
H = 8
C = D // H
K = 4

BN = 400
NB = N // BN
CH = 128
NCH = E // CH
CPT = 157
CH80 = 80
NCH80 = E // CH80
CPT80 = 250
RPT = 624
TAIL = N - 16 * RPT

_f32 = jnp.float32


@functools.cache
def _mesh():
    return plsc.VectorSubcoreMesh(core_axis_name="c", subcore_axis_name="s",
                                  num_cores=2, num_subcores=16)


def _sc_kernel(out_type, scratch_types):
    def deco(body):
        @functools.cache
        def build():
            return pl.kernel(body, out_type, mesh=_mesh(),
                             scratch_types=scratch_types)

        def call(*args):
            return build()(*args)

        return call

    return deco


def _chunk_range(w):
    start = w * CPT
    end = jnp.minimum(start + CPT, NCH)
    return start, end


def _chunk_range80(w):
    start = w * CPT80
    return start, start + CPT80


def _zero_acc(zsrc, acc, w):
    pltpu.sync_copy(zsrc, acc.at[pl.ds(w * RPT, RPT)])

    @pl.when(w == 0)
    def _():
        pltpu.sync_copy(zsrc.at[pl.ds(0, TAIL)], acc.at[pl.ds(16 * RPT, TAIL)])


def _writeout(acc, out, t, w):
    pltpu.sync_copy(acc.at[pl.ds(w * RPT, RPT)], out.at[t, pl.ds(w * RPT, RPT)])

    @pl.when(w == 0)
    def _():
        pltpu.sync_copy(acc.at[pl.ds(16 * RPT, TAIL)],
                        out.at[t, pl.ds(16 * RPT, TAIL)])


@_sc_kernel(
    jax.ShapeDtypeStruct((K, N, D), _f32),
    [
        pltpu.VMEM((3, CH), jnp.int32),
        pltpu.VMEM((CH, D), _f32),
        pltpu.VMEM_SHARED((N, D), _f32),
        pltpu.SemaphoreType.DMA((3,)),
        pltpu.SemaphoreType.DMA((2,)),
    ],
)
def _sc_degree(edges_r, zerosd, onesd, deg_out, didx, ones_v, acc, isem,
               ssem):
    c = lax.axis_index("c")
    w = lax.axis_index("s")
    pltpu.sync_copy(onesd, ones_v)
    start, end = _chunk_range(w)
    cnt = end - start
    for tt in (0, 1):
        t = c + 2 * tt
        _zero_acc(zerosd, acc, w)
        plsc.subcore_barrier()

        def idx_issue(j, t=t):
            p3 = j % 3
            pltpu.async_copy(edges_r.at[t, 1, start + j], didx.at[p3],
                             isem.at[p3])

        idx_issue(0)

        def body(j, _, t=t):
            p = j % 2
            p3 = j % 3

            @pl.when(j < cnt)
            def _():
                pltpu.make_async_copy(edges_r.at[t, 1, start + j],
                                      didx.at[p3], isem.at[p3]).wait()

            @pl.when(j >= 2)
            def _():
                pltpu.make_async_copy(ones_v, acc.at[didx.at[(j - 2) % 3]],
                                      ssem.at[p]).wait()

            @pl.when(j < cnt)
            def _():
                pltpu.async_copy(ones_v, acc.at[didx.at[p3]], ssem.at[p],
                                 add=True)

            @pl.when(j + 1 < cnt)
            def _():
                idx_issue(j + 1)

            return ()

        lax.fori_loop(0, cnt + 2, body, ())
        plsc.subcore_barrier()
        _writeout(acc, deg_out, t, w)
        plsc.subcore_barrier()


@_sc_kernel(
    jax.ShapeDtypeStruct((K, N, D), _f32),
    [
        pltpu.VMEM((3, CH), jnp.int32),
        pltpu.VMEM((3, CH), jnp.int32),
        pltpu.VMEM((2, CH, D), _f32),
        pltpu.VMEM_SHARED((N, D), _f32),
        pltpu.SemaphoreType.DMA((3,)),
        pltpu.SemaphoreType.DMA((2,)),
        pltpu.SemaphoreType.DMA((2,)),
    ],
)
def _sc_gcn_agg(mp, edges_r, zerosd, agg_out, sidx, didx, rows, acc,
                isem, gsem, ssem):
    c = lax.axis_index("c")
    w = lax.axis_index("s")
    start, end = _chunk_range(w)
    cnt = end - start
    for tt in (0, 1):
        t = c + 2 * tt
        _zero_acc(zerosd, acc, w)
        plsc.subcore_barrier()

        def idx_issue(j, t=t):
            p3 = j % 3
            ci = start + j
            pltpu.async_copy(edges_r.at[t, 0, ci], sidx.at[p3], isem.at[p3])
            pltpu.async_copy(edges_r.at[t, 1, ci], didx.at[p3], isem.at[p3])

        def idx_wait(j, t=t):
            p3 = j % 3
            ci = start + j
            pltpu.make_async_copy(edges_r.at[t, 0, ci], sidx.at[p3],
                                  isem.at[p3]).wait()
            pltpu.make_async_copy(edges_r.at[t, 1, ci], didx.at[p3],
                                  isem.at[p3]).wait()

        idx_issue(0)

        def body(j, _, t=t):
            p = j % 2
            p1 = (j + 1) % 2
            p3 = j % 3

            @pl.when(j < cnt)
            def _():
                idx_wait(j)

            @pl.when(j >= 2)
            def _():
                pltpu.make_async_copy(
                    rows.at[p], acc.at[didx.at[(j - 2) % 3]],
                    ssem.at[p]).wait()

            @pl.when(j < cnt)
            def _():
                pltpu.async_copy(mp.at[t].at[sidx.at[p3]], rows.at[p],
                                 gsem.at[p])

            @pl.when((j >= 1) & (j <= cnt))
            def _():
                pm3 = (j - 1) % 3
                pltpu.make_async_copy(
                    mp.at[t].at[sidx.at[pm3]], rows.at[p1],
                    gsem.at[p1]).wait()
                pltpu.async_copy(rows.at[p1], acc.at[didx.at[pm3]],
                                 ssem.at[p1], add=True)

            @pl.when(j + 1 < cnt)
            def _():
                idx_issue(j + 1)

            return ()

        lax.fori_loop(0, cnt + 2, body, ())
        plsc.subcore_barrier()
        _writeout(acc, agg_out, t, w)
        plsc.subcore_barrier()


@_sc_kernel(
    (
        jax.ShapeDtypeStruct((K, N, D), _f32),
        jax.ShapeDtypeStruct((K, E, D), _f32),
    ),
    [
        pltpu.VMEM((2, CH80), jnp.int32),
        pltpu.VMEM((2, CH80), jnp.int32),
        pltpu.VMEM((CH80, D), _f32),
        pltpu.VMEM((CH80, D), _f32),
        pltpu.VMEM((CH80, D), _f32),
        pltpu.VMEM((CH80, D), _f32),
        pltpu.VMEM_SHARED((N, D), _f32),
        pltpu.SemaphoreType.DMA,
        pltpu.SemaphoreType.DMA,
        pltpu.SemaphoreType.DMA,
        pltpu.SemaphoreType.DMA,
        pltpu.SemaphoreType.DMA,
        pltpu.SemaphoreType.DMA,
    ],
)
def _sc_gat1(als, ald, edges_t, zerosd, den_out, ee_out,
             eba, ebb, asba, adba, asbb, adbb, acc,
             gsa, gsb, ssa, ssb, wsa, wsb):
    c = lax.axis_index("c")
    w = lax.axis_index("s")
    start, end = _chunk_range80(w)
    npairs = (end - start) // 2

    def compute_ee(asb, adb):
        def row(r, _):
            for h in range(H):
                sl = pl.ds(16 * h, 16)
                x = asb[r, sl] + adb[r, sl]
                adb[r, sl] = jnp.exp(jnp.maximum(x, 0.2 * x))
            return ()

        lax.fori_loop(0, CH80, row, ())

    for tt in (0, 1):
        t = c + 2 * tt
        _zero_acc(zerosd, acc, w)
        plsc.subcore_barrier()

        def body(i, _, t=t):
            cia = start + 2 * i
            cib = cia + 1

            @pl.when(i >= 1)
            def _():
                pltpu.make_async_copy(adba, acc.at[eba.at[1]], ssa).wait()
                pltpu.make_async_copy(
                    adba, ee_out.at[t, pl.ds((cia - 2) * CH80, CH80)],
                    wsa).wait()

            @pl.when(i < npairs)
            def _():
                pltpu.sync_copy(edges_t.at[t, cia], eba)
                pltpu.async_copy(als.at[t].at[eba.at[0]], asba, gsa)
                pltpu.async_copy(ald.at[t].at[eba.at[1]], adba, gsa)

            @pl.when(i >= 1)
            def _():
                pltpu.make_async_copy(adbb, acc.at[ebb.at[1]], ssb).wait()
                pltpu.make_async_copy(
                    adbb, ee_out.at[t, pl.ds((cib - 2) * CH80, CH80)],
                    wsb).wait()

            @pl.when(i < npairs)
            def _():
                pltpu.sync_copy(edges_t.at[t, cib], ebb)
                pltpu.async_copy(als.at[t].at[ebb.at[0]], asbb, gsb)
                pltpu.async_copy(ald.at[t].at[ebb.at[1]], adbb, gsb)

                pltpu.make_async_copy(als.at[t].at[eba.at[0]], asba,
                                      gsa).wait()
                pltpu.make_async_copy(ald.at[t].at[eba.at[1]], adba,
                                      gsa).wait()
                compute_ee(asba, adba)
                pltpu.async_copy(adba, acc.at[eba.at[1]], ssa, add=True)
                pltpu.async_copy(adba,
                                 ee_out.at[t, pl.ds(cia * CH80, CH80)], wsa)

                pltpu.make_async_copy(als.at[t].at[ebb.at[0]], asbb,
                                      gsb).wait()
                pltpu.make_async_copy(ald.at[t].at[ebb.at[1]], adbb,
                                      gsb).wait()
                compute_ee(asbb, adbb)
                pltpu.async_copy(adbb, acc.at[ebb.at[1]], ssb, add=True)
                pltpu.async_copy(adbb,
                                 ee_out.at[t, pl.ds(cib * CH80, CH80)], wsb)

            return ()

        lax.fori_loop(0, npairs + 1, body, ())
        plsc.subcore_barrier()
        _writeout(acc, den_out, t, w)
        plsc.subcore_barrier()


@_sc_kernel(
    jax.ShapeDtypeStruct((K, N, D), _f32),
    [
        pltpu.VMEM((2, CH80), jnp.int32),
        pltpu.VMEM((2, CH80), jnp.int32),
        pltpu.VMEM((CH80, D), _f32),
        pltpu.VMEM((CH80, D), _f32),
        pltpu.VMEM((CH80, D), _f32),
        pltpu.VMEM((CH80, D), _f32),
        pltpu.VMEM_SHARED((N, D), _f32),
        pltpu.SemaphoreType.DMA,
        pltpu.SemaphoreType.DMA,
        pltpu.SemaphoreType.DMA,
        pltpu.SemaphoreType.DMA,
    ],
)
def _sc_gat2(hm, ee, edges_t, zerosd, gacc_out,
             eba, ebb, hba, eeba, hbb, eebb, acc, gsa, gsb, ssa, ssb):
    c = lax.axis_index("c")
    w = lax.axis_index("s")
    start, end = _chunk_range80(w)
    npairs = (end - start) // 2

    def compute_mul(hb, eeb):
        def row(r, _):
            for h in range(H):
                sl = pl.ds(16 * h, 16)
                eeb[r, sl] = eeb[r, sl] * hb[r, sl]
            return ()

        lax.fori_loop(0, CH80, row, ())

    for tt in (0, 1):
        t = c + 2 * tt
        _zero_acc(zerosd, acc, w)
        plsc.subcore_barrier()

        def body(i, _, t=t):
            cia = start + 2 * i
            cib = cia + 1

            @pl.when(i >= 1)
            def _():
                pltpu.make_async_copy(eeba, acc.at[eba.at[1]], ssa).wait()

            @pl.when(i < npairs)
            def _():
                pltpu.sync_copy(edges_t.at[t, cia], eba)
                pltpu.async_copy(hm.at[t].at[eba.at[0]], hba, gsa)
                pltpu.async_copy(ee.at[t, pl.ds(cia * CH80, CH80)], eeba,
                                 gsa)

            @pl.when(i >= 1)
            def _():
                pltpu.make_async_copy(eebb, acc.at[ebb.at[1]], ssb).wait()

            @pl.when(i < npairs)
            def _():
                pltpu.sync_copy(edges_t.at[t, cib], ebb)
                pltpu.async_copy(hm.at[t].at[ebb.at[0]], hbb, gsb)
                pltpu.async_copy(ee.at[t, pl.ds(cib * CH80, CH80)], eebb,
                                 gsb)

                pltpu.make_async_copy(hm.at[t].at[eba.at[0]], hba,
                                      gsa).wait()
                pltpu.make_async_copy(ee.at[t, pl.ds(cia * CH80, CH80)],
                                      eeba, gsa).wait()
                compute_mul(hba, eeba)
                pltpu.async_copy(eeba, acc.at[eba.at[1]], ssa, add=True)

                pltpu.make_async_copy(hm.at[t].at[ebb.at[0]], hbb,
                                      gsb).wait()
                pltpu.make_async_copy(ee.at[t, pl.ds(cib * CH80, CH80)],
                                      eebb, gsb).wait()
                compute_mul(hbb, eebb)
                pltpu.async_copy(eebb, acc.at[ebb.at[1]], ssb, add=True)

            return ()

        lax.fori_loop(0, npairs + 1, body, ())
        plsc.subcore_barrier()
        _writeout(acc, gacc_out, t, w)
        plsc.subcore_barrier()


def _dinv_of(deg_ref):
    return lax.rsqrt(deg_ref[0, :, 0:1] + 1.0)


def _tc0_body(x_ref, w_ref, deg_ref, m_ref, mp_ref):
    m = jnp.dot(x_ref[...], w_ref[0].T, preferred_element_type=_f32)
    dinv = _dinv_of(deg_ref)
    m_ref[0] = m
    mp_ref[0] = m * dinv


def _tc_layer_body(agg_ref, mprev_ref, deg_ref, b_ref, w_ref, m_ref, mp_ref):
    dinv = _dinv_of(deg_ref)
    h = jnp.maximum(dinv * agg_ref[0] + dinv * dinv * mprev_ref[0] + b_ref[0],
                    0.0)
    m = jnp.dot(h, w_ref[0].T, preferred_element_type=_f32)
    m_ref[0] = m
    mp_ref[0] = m * dinv


def _tc_gatprep_body(agg_ref, mprev_ref, deg_ref, b_ref, w_ref, asx_ref,
                     adx_ref, hm_ref, als_ref, ald_ref, ees_ref):
    dinv = _dinv_of(deg_ref)
    h = jnp.maximum(dinv * agg_ref[0] + dinv * dinv * mprev_ref[0] + b_ref[0],
                    0.0)
    hmv = jnp.dot(h, w_ref[0].T, preferred_element_type=_f32)
    als = jnp.dot(hmv, asx_ref[0], preferred_element_type=_f32)
    ald = jnp.dot(hmv, adx_ref[0], preferred_element_type=_f32)
    x = als + ald
    ee = jnp.exp(jnp.maximum(x, 0.2 * x))
    hm_ref[0] = hmv
    als_ref[0] = als
    ald_ref[0] = ald
    ees_ref[0] = ee


def _tc_final_body(gacc_ref, hm_ref, ees_ref, den_ref, gb_ref, lng_ref,
                   lnb_ref, sty_ref, strs_ref, w1a_ref, w1b_ref,
                   b1_ref, w2_ref, b2_ref, out_ref):
    a = jnp.tanh(jnp.dot(sty_ref[...], w1a_ref[...].T,
                         preferred_element_type=_f32)
                 + jnp.dot(strs_ref[...], w1b_ref[...].T,
                           preferred_element_type=_f32)
                 + b1_ref[...])
    logits = jnp.dot(a, w2_ref[...], preferred_element_type=_f32) + b2_ref[...]
    mx = jnp.max(logits, axis=-1, keepdims=True)
    ew = jnp.exp(logits - mx)
    wgt = ew / jnp.sum(ew, axis=-1, keepdims=True)
    acc = jnp.zeros((BN, D), _f32)
    for k in range(K):
        din = 1.0 / (den_ref[k] + ees_ref[k] + 1e-16)
        g = din * (gacc_ref[k] + ees_ref[k] * hm_ref[k]) + gb_ref[k]
        mu = jnp.mean(g, axis=-1, keepdims=True)
        var = jnp.mean((g - mu) ** 2, axis=-1, keepdims=True)
        hk = (g - mu) * lax.rsqrt(var + 1e-5) * lng_ref[k] + lnb_ref[k]
        acc = acc + wgt[:, k:k + 1] * hk
    out_ref[...] = acc


def _nd(shape):
    return jax.ShapeDtypeStruct(shape, _f32)


_B_KND = pl.BlockSpec((1, BN, D), lambda k, i: (k, i, 0))
_B_KDD = pl.BlockSpec((1, D, D), lambda k, i: (k, 0, 0))
_B_K1D = pl.BlockSpec((1, 1, D), lambda k, i: (k, 0, 0))


def _tc0(x, w0, deg):
    return pl.pallas_call(
        _tc0_body,
        grid=(K, NB),
        in_specs=[pl.BlockSpec((BN, D), lambda k, i: (i, 0)), _B_KDD, _B_KND],
        out_specs=[_B_KND, _B_KND],
        out_shape=[_nd((K, N, D)), _nd((K, N, D))],
    )(x, w0, deg)


def _tc_layer(agg, mprev, deg, b, w):
    return pl.pallas_call(
        _tc_layer_body,
        grid=(K, NB),
        in_specs=[_B_KND, _B_KND, _B_KND, _B_K1D, _B_KDD],
        out_specs=[_B_KND, _B_KND],
        out_shape=[_nd((K, N, D)), _nd((K, N, D))],
    )(agg, mprev, deg, b, w)


def _tc_gatprep(agg, mprev, deg, b, w, asx, adx):
    return pl.pallas_call(
        _tc_gatprep_body,
        grid=(K, NB),
        in_specs=[_B_KND, _B_KND, _B_KND, _B_K1D, _B_KDD, _B_KDD, _B_KDD],
        out_specs=[_B_KND, _B_KND, _B_KND, _B_KND],
        out_shape=[_nd((K, N, D)), _nd((K, N, D)), _nd((K, N, D)),
                   _nd((K, N, D))],
    )(agg, mprev, deg, b, w, asx, adx)


def _tc_final(gacc, hm, ees, den, gat_b, ln_g, ln_b, style, stress,
              w1a, w1b, b1, w2p, b2p):
    full = lambda *s: pl.BlockSpec(s, lambda i: (0,) * len(s))
    return pl.pallas_call(
        _tc_final_body,
        grid=(NB,),
        in_specs=[
            pl.BlockSpec((K, BN, D), lambda i: (0, i, 0)),
            pl.BlockSpec((K, BN, D), lambda i: (0, i, 0)),
            pl.BlockSpec((K, BN, D), lambda i: (0, i, 0)),
            pl.BlockSpec((K, BN, D), lambda i: (0, i, 0)),
            full(K, D), full(K, D), full(K, D),
            pl.BlockSpec((BN, D), lambda i: (i, 0)),
            pl.BlockSpec((BN, D), lambda i: (i, 0)),
            full(D, D), full(D, D), full(D), full(D, 8), full(8),
        ],
        out_specs=pl.BlockSpec((BN, D), lambda i: (i, 0)),
        out_shape=_nd((N, D)),
    )(gacc, hm, ees, den, gat_b, ln_g, ln_b, style, stress,
      w1a, w1b, b1, w2p, b2p)


def kernel(x, edge_index_list, style_features, stress_features, gcn_W, gcn_b,
           gat_W, gat_att_src, gat_att_dst, gat_b, ln_g, ln_b, fus_W1, fus_b1,
           fus_W2, fus_b2):
    edges_r = edge_index_list.reshape(K, 2, NCH, CH)
    edges_t80 = edge_index_list.reshape(K, 2, NCH80, CH80).transpose(0, 2, 1, 3)
    zerosd = jnp.zeros((RPT, D), _f32)
    onesd = jnp.ones((CH, D), _f32)

    rep = jnp.kron(jnp.eye(H, dtype=_f32), jnp.ones((1, C), _f32))
    asbd = jnp.einsum('khc,hg->khcg', gat_att_src,
                      jnp.eye(H, dtype=_f32)).reshape(K, D, H)
    adbd = jnp.einsum('khc,hg->khcg', gat_att_dst,
                      jnp.eye(H, dtype=_f32)).reshape(K, D, H)
    asx = jnp.einsum('kdh,hg->kdg', asbd, rep)
    adx = jnp.einsum('kdh,hg->kdg', adbd, rep)
    w1a = fus_W1[:, :D]
    w1b = fus_W1[:, D:]
    w2p = jnp.pad(fus_W2.T, ((0, 0), (0, 8 - K)))
    b2p = jnp.concatenate([fus_b2, jnp.full((8 - K,), -1e30, _f32)])

    deg = _sc_degree(edges_r, zerosd, onesd)

    m, mp = _tc0(x, gcn_W[:, 0], deg)
    for l in (1, 2):
        agg = _sc_gcn_agg(mp, edges_r, zerosd)
        m, mp = _tc_layer(agg, m, deg, gcn_b[:, l - 1][:, None, :],
                          gcn_W[:, l])
    agg = _sc_gcn_agg(mp, edges_r, zerosd)
    hm, als, ald, ees = _tc_gatprep(agg, m, deg, gcn_b[:, 2][:, None, :],
                                    gat_W, asx, adx)
    den, ee = _sc_gat1(als, ald, edges_t80, zerosd)
    gacc = _sc_gat2(hm, ee, edges_t80, zerosd)
    return _tc_final(gacc, hm, ees, den, gat_b, ln_g, ln_b,
                     style_features, stress_features, w1a, w1b, fus_b1,
                     w2p, b2p)
